# baseline (speedup 1.0000x reference)
import functools
import jax
import jax.numpy as jnp
from jax import lax
from jax.experimental import pallas as pl
from jax.experimental.pallas import tpu as pltpu
from jax.experimental.pallas import tpu_sc as plsc

_NC, _NF, _DX, _DS, _B = 4096, 16384, 64, 64, 16
_DIN, _DOUT, _K = 128, 128, 3
_T = 512
_NT = _NF // _T
_W = 256
_INF = float("inf")

_NW = 32
_PW = _NF // _NW
_PC = 256
_NCH_SC = _PW // _PC
_G = 16


def _knn_body(cbase_ref, nch_ref, pos_t_ref, batch_row_ref,
              pos_skip_ref, batch_col_ref,
              i1_ref, i2_ref, i3_ref, w1_ref, w2_ref, w3_ref):
    t = pl.program_id(0)
    if True:
        ps = pos_skip_ref[...]
        bs = batch_col_ref[...]
        psn = jnp.sum(ps * ps, axis=1, keepdims=True)
        ps_bf = ps.astype(jnp.bfloat16)
        base0 = cbase_ref[t]
        nch = nch_ref[t]

        def p1(c, carry):
            v1, i1, v2, i2, v3, i3 = carry
            cb = pl.multiple_of(base0 + c * _W, _W)
            pt = pos_t_ref[:, pl.ds(cb, _W)]
            br = batch_row_ref[:, pl.ds(cb, _W)]
            cross = jnp.dot(ps_bf, pt.astype(jnp.bfloat16),
                            preferred_element_type=jnp.float32)
            pn = jnp.sum(pt * pt, axis=0, keepdims=True)
            d2 = jnp.maximum((psn + pn) - 2.0 * cross, 0.0)
            d2 = jnp.where(bs != br, _INF, d2)
            giota = cb + lax.broadcasted_iota(jnp.int32, (1, _W), 1)
            cand = [v1, i1, v2, i2, v3, i3]
            for _j in range(_K):
                m = jnp.min(d2, axis=1, keepdims=True)
                g = jnp.min(jnp.where(d2 <= m, giota, _NC),
                            axis=1, keepdims=True)
                d2 = jnp.where(giota == g, _INF, d2)
                cand.extend([m, g])
            vs = cand[0::2]
            gs = cand[1::2]
            out = []
            for _j in range(_K):
                mv = vs[0]
                for v in vs[1:]:
                    mv = jnp.minimum(mv, v)
                mi = jnp.full((_T, 1), _NC, jnp.int32)
                for v, g in zip(vs, gs):
                    mi = jnp.minimum(mi, jnp.where(v <= mv, g, _NC))
                out.extend([mv, mi])
                vs = [jnp.where((v <= mv) & (g == mi), _INF, v)
                      for v, g in zip(vs, gs)]
            return tuple(out)

        ful = jnp.full((_T, 1), _INF, jnp.float32)
        zi = jnp.zeros((_T, 1), jnp.int32)
        v1, i1, v2, i2, v3, i3 = lax.fori_loop(
            0, nch, p1, (ful, zi, ful, zi, ful, zi))
        w1 = 1.0 / jnp.maximum(v1, 1e-16)
        w2 = 1.0 / jnp.maximum(v2, 1e-16)
        w3 = 1.0 / jnp.maximum(v3, 1e-16)
        inv = 1.0 / (w1 + w2 + w3)
        i1_ref[...] = i1
        i2_ref[...] = i2
        i3_ref[...] = i3
        w1_ref[...] = w1 * inv
        w2_ref[...] = w2 * inv
        w3_ref[...] = w3 * inv


def _sc_interp(x_hbm, i1_hbm, i2_hbm, i3_hbm, w1_hbm, w2_hbm, w3_hbm,
               y_hbm, i1_v, i2_v, i3_v, w1_v, w2_v, w3_v,
               r1_v, r2_v, r3_v, y_v, sem):
    wid = lax.axis_index("s") * 2 + lax.axis_index("c")

    def chunk(ch, _):
        pbase = wid * _PW + ch * _PC
        for ih, iv in ((i1_hbm, i1_v), (i2_hbm, i2_v), (i3_hbm, i3_v)):
            pltpu.sync_copy(ih.at[pl.ds(pbase, _PC)], iv)
        for wh, wv in ((w1_hbm, w1_v), (w2_hbm, w2_v), (w3_hbm, w3_v)):
            pltpu.sync_copy(wh.at[pl.ds(pbase, _PC)], wv)
        for iv, rv in ((i1_v, r1_v), (i2_v, r2_v), (i3_v, r3_v)):
            pltpu.async_copy(x_hbm.at[iv], rv, sem).wait()

        def group(gi, _):
            gb = pl.multiple_of(gi * _G, _G)
            w1g = w1_v[pl.ds(gb, _G)]
            w2g = w2_v[pl.ds(gb, _G)]
            w3g = w3_v[pl.ds(gb, _G)]
            for l in range(_G):
                p = gb + l
                for cblk in range(_DX // 16):
                    sl = pl.ds(cblk * 16, 16)
                    y_v[pl.ds(p * _DX + cblk * 16, 16)] = (
                        w1g[l] * r1_v[p, sl]
                        + w2g[l] * r2_v[p, sl]
                        + w3g[l] * r3_v[p, sl])
            return 0

        lax.fori_loop(0, _PC // _G, group, 0)
        pltpu.sync_copy(y_v, y_hbm.at[pl.ds(pbase * _DX, _PC * _DX)])
        return 0

    lax.fori_loop(0, _NCH_SC, chunk, 0)


def _mlp_body(y_ref, x_skip_ref, w1a_ref, w1b_ref, p1_ref, w2_ref, p2_ref,
              out_ref, h1_ref):
    def tile_mlp1(t, carry):
        s1, q1 = carry
        o = pl.multiple_of(t * _T, _T)
        y = y_ref[pl.ds(o, _T), :]
        xs = x_skip_ref[pl.ds(o, _T), :]
        h1 = (jnp.dot(y, w1a_ref[...], preferred_element_type=jnp.float32)
              + jnp.dot(xs, w1b_ref[...], preferred_element_type=jnp.float32)
              + p1_ref[0:1, :])
        h1_ref[pl.ds(o, _T), :] = h1
        return (s1 + jnp.sum(h1, axis=0, keepdims=True),
                q1 + jnp.sum(h1 * h1, axis=0, keepdims=True))

    z = jnp.zeros((1, _DOUT), jnp.float32)
    s1, q1 = lax.fori_loop(0, _NT, tile_mlp1, (z, z))
    mu1 = s1 / _NF
    var1 = q1 / _NF - mu1 * mu1
    sc1 = p1_ref[1:2, :] * lax.rsqrt(var1 + 1e-5)
    sh1 = p1_ref[2:3, :] - mu1 * sc1

    def tile_mlp2(t, carry):
        s2, q2 = carry
        o = pl.multiple_of(t * _T, _T)
        h1 = h1_ref[pl.ds(o, _T), :]
        zrel = jnp.maximum(h1 * sc1 + sh1, 0.0)
        h2 = (jnp.dot(zrel, w2_ref[...], preferred_element_type=jnp.float32)
              + p2_ref[0:1, :])
        out_ref[pl.ds(o, _T), :] = h2
        return (s2 + jnp.sum(h2, axis=0, keepdims=True),
                q2 + jnp.sum(h2 * h2, axis=0, keepdims=True))

    s2, q2 = lax.fori_loop(0, _NT, tile_mlp2, (z, z))
    mu2 = s2 / _NF
    var2 = q2 / _NF - mu2 * mu2
    sc2 = p2_ref[1:2, :] * lax.rsqrt(var2 + 1e-5)
    sh2 = p2_ref[2:3, :] - mu2 * sc2

    def tile_bn2(t, _):
        o = pl.multiple_of(t * _T, _T)
        h2 = out_ref[pl.ds(o, _T), :]
        out_ref[pl.ds(o, _T), :] = jnp.maximum(h2 * sc2 + sh2, 0.0)
        return 0

    lax.fori_loop(0, _NT, tile_bn2, 0)


def kernel(x, pos, batch, x_skip, pos_skip, batch_skip,
           W1, b1, g1, be1, W2, b2, g2, be2):
    pos_t = pos.T
    batch_i = batch.astype(jnp.int32)
    batch_row = batch_i.reshape(1, _NC)
    batch_col = batch_skip.astype(jnp.int32).reshape(_NF, 1)
    w1a, w1b = W1[:_DX], W1[_DX:]
    p1 = jnp.stack([b1, g1, be1])
    p2 = jnp.stack([b2, g2, be2])

    tids = jnp.arange(_NT)
    blo = batch_skip[tids * _T]
    bhi = batch_skip[tids * _T + (_T - 1)]
    clo = jnp.searchsorted(batch_i, blo.astype(jnp.int32), side="left")
    chi = jnp.searchsorted(batch_i, bhi.astype(jnp.int32), side="right")
    cbase = ((clo // _W) * _W).astype(jnp.int32)
    nch = ((chi.astype(jnp.int32) - cbase + _W - 1) // _W)

    grid_spec = pltpu.PrefetchScalarGridSpec(
        num_scalar_prefetch=2,
        grid=(_NT,),
        in_specs=[
            pl.BlockSpec((3, _NC), lambda t, cb, nc: (0, 0)),
            pl.BlockSpec((1, _NC), lambda t, cb, nc: (0, 0)),
            pl.BlockSpec((_T, 3), lambda t, cb, nc: (t, 0)),
            pl.BlockSpec((_T, 1), lambda t, cb, nc: (t, 0)),
        ],
        out_specs=[pl.BlockSpec((_T, 1), lambda t, cb, nc: (t, 0))] * 6,
    )
    cols = pl.pallas_call(
        _knn_body,
        grid_spec=grid_spec,
        out_shape=[jax.ShapeDtypeStruct((_NF, 1), jnp.int32)] * 3
        + [jax.ShapeDtypeStruct((_NF, 1), jnp.float32)] * 3,
    )(cbase, nch, pos_t, batch_row, pos_skip, batch_col)
    i1, i2, i3, wn1, wn2, wn3 = [c.reshape(_NF) for c in cols]

    mesh = plsc.VectorSubcoreMesh(core_axis_name="c", subcore_axis_name="s")
    sc = functools.partial(
        pl.kernel, mesh=mesh,
        out_type=jax.ShapeDtypeStruct((_NF * _DX,), jnp.float32),
        scratch_types=[
            pltpu.VMEM((_PC,), jnp.int32),
            pltpu.VMEM((_PC,), jnp.int32),
            pltpu.VMEM((_PC,), jnp.int32),
            pltpu.VMEM((_PC,), jnp.float32),
            pltpu.VMEM((_PC,), jnp.float32),
            pltpu.VMEM((_PC,), jnp.float32),
            pltpu.VMEM((_PC, 2 * _DX), jnp.float32),
            pltpu.VMEM((_PC, 2 * _DX), jnp.float32),
            pltpu.VMEM((_PC, 2 * _DX), jnp.float32),
            pltpu.VMEM((_PC * _DX,), jnp.float32),
            pltpu.SemaphoreType.DMA,
        ],
    )(_sc_interp)
    x_pad = jnp.concatenate([x, jnp.zeros_like(x)], axis=1)
    y = sc(x_pad, i1, i2, i3, wn1, wn2, wn3).reshape(_NF, _DX)

    h = pl.pallas_call(
        _mlp_body,
        out_shape=jax.ShapeDtypeStruct((_NF, _DOUT), jnp.float32),
        scratch_shapes=[pltpu.VMEM((_NF, _DOUT), jnp.float32)],
    )(y, x_skip, w1a, w1b, p1, W2, p2)
    return (h, pos_skip, batch_skip)


# speedup vs baseline: 2.0875x; 2.0875x over previous
import jax
import jax.numpy as jnp
from jax import lax
from jax.experimental import pallas as pl
from jax.experimental.pallas import tpu as pltpu

_NC, _NF, _DX, _DS, _B = 4096, 16384, 64, 64, 16
_DIN, _DOUT, _K = 128, 128, 3
_T = 512
_NT = _NF // _T
_W = 256
_MAXCH = _NC // _W
_INF = float("inf")


def _body(pos_t_ref, batch_row_ref, x_hi_ref, x_lo_ref, pos_skip_ref,
          batch_col_ref,
          x_skip_ref, w1a_ref, w1b_ref, p1_ref, w2_ref, p2_ref,
          cbase_ref, nch_ref, out_ref, h1_ref, d2c_ref):

    def tile_knn_mlp1(t, carry):
        s1, q1 = carry
        o = pl.multiple_of(t * _T, _T)
        ps = pos_skip_ref[pl.ds(o, _T), :]
        bs = batch_col_ref[pl.ds(o, _T), :]
        psn = jnp.sum(ps * ps, axis=1, keepdims=True)
        ps_bf = ps.astype(jnp.bfloat16)
        base0 = cbase_ref[t]
        nch = nch_ref[t]

        def d2_chunk(c):
            cb = pl.multiple_of(base0 + c * _W, _W)
            pt = pos_t_ref[:, pl.ds(cb, _W)]
            br = batch_row_ref[:, pl.ds(cb, _W)]
            cross = jnp.dot(ps_bf, pt.astype(jnp.bfloat16),
                            preferred_element_type=jnp.float32)
            pn = jnp.sum(pt * pt, axis=0, keepdims=True)
            d2 = jnp.maximum((psn + pn) - 2.0 * cross, 0.0)
            return jnp.where(bs != br, _INF, d2), cb

        def p1(c, vs):
            d2, _ = d2_chunk(c)
            d2c_ref[:, pl.ds(c * _W, _W)] = d2
            m1 = jnp.min(d2, axis=1, keepdims=True)
            d2 = jnp.where(d2 <= m1, _INF, d2)
            m2 = jnp.min(d2, axis=1, keepdims=True)
            d2 = jnp.where(d2 <= m2, _INF, d2)
            m3 = jnp.min(d2, axis=1, keepdims=True)
            cur = list(vs) + [m1, m2, m3]
            out = []
            for _ in range(_K):
                m = cur[0]
                for v in cur[1:]:
                    m = jnp.minimum(m, v)
                out.append(m)
                cur = [jnp.where(v <= m, _INF, v) for v in cur]
            return tuple(out)

        ful = jnp.full((_T, 1), _INF, jnp.float32)
        _, _, v3 = lax.fori_loop(0, nch, p1, (ful, ful, ful))

        def p2(c, acc):
            ynum, den = acc
            cb = pl.multiple_of(base0 + c * _W, _W)
            d2 = d2c_ref[:, pl.ds(c * _W, _W)]
            w = jnp.where(d2 <= v3, 1.0 / jnp.maximum(d2, 1e-16), 0.0)
            den = den + jnp.sum(w, axis=1, keepdims=True)
            w_hi = w.astype(jnp.bfloat16)
            w_lo = (w - w_hi.astype(jnp.float32)).astype(jnp.bfloat16)
            xh = x_hi_ref[pl.ds(cb, _W), :]
            xl = x_lo_ref[pl.ds(cb, _W), :]
            ynum = (ynum
                    + jnp.dot(w_hi, xh, preferred_element_type=jnp.float32)
                    + jnp.dot(w_hi, xl, preferred_element_type=jnp.float32)
                    + jnp.dot(w_lo, xh, preferred_element_type=jnp.float32))
            return ynum, den

        ynum, den = lax.fori_loop(
            0, nch, p2, (jnp.zeros((_T, _DX), jnp.float32),
                         jnp.zeros((_T, 1), jnp.float32)))
        y = ynum / den

        xs = x_skip_ref[pl.ds(o, _T), :]
        h1 = (jnp.dot(y, w1a_ref[...], preferred_element_type=jnp.float32)
              + jnp.dot(xs, w1b_ref[...], preferred_element_type=jnp.float32)
              + p1_ref[0:1, :])
        h1_ref[pl.ds(o, _T), :] = h1
        return (s1 + jnp.sum(h1, axis=0, keepdims=True),
                q1 + jnp.sum(h1 * h1, axis=0, keepdims=True))

    z = jnp.zeros((1, _DOUT), jnp.float32)
    s1, q1 = lax.fori_loop(0, _NT, tile_knn_mlp1, (z, z))
    mu1 = s1 / _NF
    var1 = q1 / _NF - mu1 * mu1
    sc1 = p1_ref[1:2, :] * lax.rsqrt(var1 + 1e-5)
    sh1 = p1_ref[2:3, :] - mu1 * sc1

    def tile_mlp2(t, carry):
        s2, q2 = carry
        o = pl.multiple_of(t * _T, _T)
        h1 = h1_ref[pl.ds(o, _T), :]
        zrel = jnp.maximum(h1 * sc1 + sh1, 0.0)
        h2 = (jnp.dot(zrel, w2_ref[...], preferred_element_type=jnp.float32)
              + p2_ref[0:1, :])
        out_ref[pl.ds(o, _T), :] = h2
        return (s2 + jnp.sum(h2, axis=0, keepdims=True),
                q2 + jnp.sum(h2 * h2, axis=0, keepdims=True))

    s2, q2 = lax.fori_loop(0, _NT, tile_mlp2, (z, z))
    mu2 = s2 / _NF
    var2 = q2 / _NF - mu2 * mu2
    sc2 = p2_ref[1:2, :] * lax.rsqrt(var2 + 1e-5)
    sh2 = p2_ref[2:3, :] - mu2 * sc2

    def tile_bn2(t, _):
        o = pl.multiple_of(t * _T, _T)
        h2 = out_ref[pl.ds(o, _T), :]
        out_ref[pl.ds(o, _T), :] = jnp.maximum(h2 * sc2 + sh2, 0.0)
        return 0

    lax.fori_loop(0, _NT, tile_bn2, 0)


def kernel(x, pos, batch, x_skip, pos_skip, batch_skip,
           W1, b1, g1, be1, W2, b2, g2, be2):
    pos_t = pos.T
    batch_i = batch.astype(jnp.int32)
    batch_row = batch_i.reshape(1, _NC)
    batch_col = batch_skip.astype(jnp.int32).reshape(_NF, 1)
    x_hi = x.astype(jnp.bfloat16)
    x_lo = (x - x_hi.astype(jnp.float32)).astype(jnp.bfloat16)
    w1a, w1b = W1[:_DX], W1[_DX:]
    p1 = jnp.stack([b1, g1, be1])
    p2 = jnp.stack([b2, g2, be2])

    tids = jnp.arange(_NT)
    blo = batch_skip[tids * _T]
    bhi = batch_skip[tids * _T + (_T - 1)]
    clo = jnp.searchsorted(batch_i, blo.astype(jnp.int32), side="left")
    chi = jnp.searchsorted(batch_i, bhi.astype(jnp.int32), side="right")
    cbase = ((clo // _W) * _W).astype(jnp.int32)
    nch = ((chi.astype(jnp.int32) - cbase + _W - 1) // _W)

    h = pl.pallas_call(
        _body,
        out_shape=jax.ShapeDtypeStruct((_NF, _DOUT), jnp.float32),
        in_specs=[pl.BlockSpec(memory_space=pltpu.VMEM)] * 12
        + [pl.BlockSpec(memory_space=pltpu.SMEM)] * 2,
        out_specs=pl.BlockSpec(memory_space=pltpu.VMEM),
        scratch_shapes=[pltpu.VMEM((_NF, _DOUT), jnp.float32),
                        pltpu.VMEM((_T, _MAXCH * _W), jnp.float32)],
    )(pos_t, batch_row, x_hi, x_lo, pos_skip, batch_col, x_skip,
      w1a, w1b, p1, W2, p2, cbase, nch)
    return (h, pos_skip, batch_skip)


# speedup vs baseline: 2.3535x; 1.1274x over previous
import jax
import jax.numpy as jnp
from jax import lax
from jax.experimental import pallas as pl
from jax.experimental.pallas import tpu as pltpu

_NC, _NF, _DX, _DS, _B = 4096, 16384, 64, 64, 16
_DIN, _DOUT, _K = 128, 128, 3
_T = 512
_NT = _NF // _T
_W = 256
_MAXCH = _NC // _W
_INF = float("inf")


def _body(pos_t_ref, batch_row_ref, x_hi_ref, x_lo_ref, pos_skip_ref,
          batch_col_ref,
          x_skip_ref, w1a_ref, w1b_ref, p1_ref, w2_ref, p2_ref,
          cbase_ref, nch_ref, out_ref, h1_ref, d2c_ref):

    def tile_knn_mlp1(t, carry):
        s1, q1 = carry
        o = pl.multiple_of(t * _T, _T)
        ps = pos_skip_ref[pl.ds(o, _T), :]
        bs = batch_col_ref[pl.ds(o, _T), :]
        psn = jnp.sum(ps * ps, axis=1, keepdims=True)
        ps_bf = ps.astype(jnp.bfloat16)
        base0 = cbase_ref[t]
        nch = nch_ref[t]

        def d2_chunk(c):
            cb = pl.multiple_of(base0 + c * _W, _W)
            pt = pos_t_ref[:, pl.ds(cb, _W)]
            br = batch_row_ref[:, pl.ds(cb, _W)]
            cross = jnp.dot(ps_bf, pt.astype(jnp.bfloat16),
                            preferred_element_type=jnp.float32)
            pn = jnp.sum(pt * pt, axis=0, keepdims=True)
            d2 = jnp.maximum((psn + pn) - 2.0 * cross, 0.0)
            return jnp.where(bs != br, _INF, d2), cb

        def p1(c, vs):
            d2, _ = d2_chunk(c)
            d2c_ref[:, pl.ds(c * _W, _W)] = d2
            m1 = jnp.min(d2, axis=1, keepdims=True)
            d2 = jnp.where(d2 <= m1, _INF, d2)
            m2 = jnp.min(d2, axis=1, keepdims=True)
            d2 = jnp.where(d2 <= m2, _INF, d2)
            m3 = jnp.min(d2, axis=1, keepdims=True)
            v1, v2, v3 = vs
            r1 = jnp.minimum(v1, m1)
            b1 = jnp.maximum(v1, m1)
            a2 = jnp.minimum(v2, m2)
            r2 = jnp.minimum(b1, a2)
            r3 = jnp.minimum(jnp.maximum(b1, a2),
                             jnp.minimum(v3, m3))
            return (r1, r2, r3)

        ful = jnp.full((_T, 1), _INF, jnp.float32)
        _, _, v3 = lax.fori_loop(0, nch, p1, (ful, ful, ful))

        def p2(c, acc):
            ynum, den = acc
            cb = pl.multiple_of(base0 + c * _W, _W)
            d2 = d2c_ref[:, pl.ds(c * _W, _W)]
            w = jnp.where(d2 <= v3, 1.0 / jnp.maximum(d2, 1e-16), 0.0)
            den = den + jnp.sum(w, axis=1, keepdims=True)
            w_hi = w.astype(jnp.bfloat16)
            w_lo = (w - w_hi.astype(jnp.float32)).astype(jnp.bfloat16)
            xh = x_hi_ref[pl.ds(cb, _W), :]
            xl = x_lo_ref[pl.ds(cb, _W), :]
            ynum = (ynum
                    + jnp.dot(w_hi, xh, preferred_element_type=jnp.float32)
                    + jnp.dot(w_hi, xl, preferred_element_type=jnp.float32)
                    + jnp.dot(w_lo, xh, preferred_element_type=jnp.float32))
            return ynum, den

        ynum, den = lax.fori_loop(
            0, nch, p2, (jnp.zeros((_T, _DX), jnp.float32),
                         jnp.zeros((_T, 1), jnp.float32)))
        y = ynum / den

        xs = x_skip_ref[pl.ds(o, _T), :]
        h1 = (jnp.dot(y, w1a_ref[...], preferred_element_type=jnp.float32)
              + jnp.dot(xs, w1b_ref[...], preferred_element_type=jnp.float32)
              + p1_ref[0:1, :])
        h1_ref[pl.ds(o, _T), :] = h1
        return (s1 + jnp.sum(h1, axis=0, keepdims=True),
                q1 + jnp.sum(h1 * h1, axis=0, keepdims=True))

    z = jnp.zeros((1, _DOUT), jnp.float32)
    s1, q1 = lax.fori_loop(0, _NT, tile_knn_mlp1, (z, z))
    mu1 = s1 / _NF
    var1 = q1 / _NF - mu1 * mu1
    sc1 = p1_ref[1:2, :] * lax.rsqrt(var1 + 1e-5)
    sh1 = p1_ref[2:3, :] - mu1 * sc1

    def tile_mlp2(t, carry):
        s2, q2 = carry
        o = pl.multiple_of(t * _T, _T)
        h1 = h1_ref[pl.ds(o, _T), :]
        zrel = jnp.maximum(h1 * sc1 + sh1, 0.0)
        h2 = (jnp.dot(zrel, w2_ref[...], preferred_element_type=jnp.float32)
              + p2_ref[0:1, :])
        out_ref[pl.ds(o, _T), :] = h2
        return (s2 + jnp.sum(h2, axis=0, keepdims=True),
                q2 + jnp.sum(h2 * h2, axis=0, keepdims=True))

    s2, q2 = lax.fori_loop(0, _NT, tile_mlp2, (z, z))
    mu2 = s2 / _NF
    var2 = q2 / _NF - mu2 * mu2
    sc2 = p2_ref[1:2, :] * lax.rsqrt(var2 + 1e-5)
    sh2 = p2_ref[2:3, :] - mu2 * sc2

    def tile_bn2(t, _):
        o = pl.multiple_of(t * _T, _T)
        h2 = out_ref[pl.ds(o, _T), :]
        out_ref[pl.ds(o, _T), :] = jnp.maximum(h2 * sc2 + sh2, 0.0)
        return 0

    lax.fori_loop(0, _NT, tile_bn2, 0)


def kernel(x, pos, batch, x_skip, pos_skip, batch_skip,
           W1, b1, g1, be1, W2, b2, g2, be2):
    pos_t = pos.T
    batch_i = batch.astype(jnp.int32)
    batch_row = batch_i.reshape(1, _NC)
    batch_col = batch_skip.astype(jnp.int32).reshape(_NF, 1)
    x_hi = x.astype(jnp.bfloat16)
    x_lo = (x - x_hi.astype(jnp.float32)).astype(jnp.bfloat16)
    w1a, w1b = W1[:_DX], W1[_DX:]
    p1 = jnp.stack([b1, g1, be1])
    p2 = jnp.stack([b2, g2, be2])

    tids = jnp.arange(_NT)
    blo = batch_skip[tids * _T]
    bhi = batch_skip[tids * _T + (_T - 1)]
    clo = jnp.searchsorted(batch_i, blo.astype(jnp.int32), side="left")
    chi = jnp.searchsorted(batch_i, bhi.astype(jnp.int32), side="right")
    cbase = ((clo // _W) * _W).astype(jnp.int32)
    nch = ((chi.astype(jnp.int32) - cbase + _W - 1) // _W)

    h = pl.pallas_call(
        _body,
        out_shape=jax.ShapeDtypeStruct((_NF, _DOUT), jnp.float32),
        in_specs=[pl.BlockSpec(memory_space=pltpu.VMEM)] * 12
        + [pl.BlockSpec(memory_space=pltpu.SMEM)] * 2,
        out_specs=pl.BlockSpec(memory_space=pltpu.VMEM),
        scratch_shapes=[pltpu.VMEM((_NF, _DOUT), jnp.float32),
                        pltpu.VMEM((_T, _MAXCH * _W), jnp.float32)],
    )(pos_t, batch_row, x_hi, x_lo, pos_skip, batch_col, x_skip,
      w1a, w1b, p1, W2, p2, cbase, nch)
    return (h, pos_skip, batch_skip)
